# initial kernel scaffold (unmeasured)
import jax
import jax.numpy as jnp
from jax import lax
from jax.experimental import pallas as pl
from jax.experimental.pallas import tpu as pltpu

N_DEV = 16
B = 2
SQ = 256
D_MODEL = 512
HQ = 4
DH = 64
HD = HQ * DH
BLK = 64
NR = SQ // BLK


def kernel(x, Wq, K_ext, V_ext, Wo):
    def body(x_ref, wq_ref, k_ref, v_ref, wo_ref, out_ref,
             kv_buf, ctx_buf, send_sems, recv_sems):
        my = lax.axis_index("i")
        left = lax.rem(my + N_DEV - 1, N_DEV)
        right = lax.rem(my + 1, N_DEV)

        barrier_sem = pltpu.get_barrier_semaphore()
        for nbr in (left, right):
            pl.semaphore_signal(
                barrier_sem, inc=1,
                device_id=(nbr,), device_id_type=pl.DeviceIdType.MESH,
            )
        pl.semaphore_wait(barrier_sem, 2)

        kv_buf[0, :, :, 0:HD] = k_ref[...].reshape(B, SQ, HD).astype(jnp.bfloat16)
        kv_buf[0, :, :, HD:2 * HD] = v_ref[...].reshape(B, SQ, HD).astype(jnp.bfloat16)

        for h in range(N_DEV - 1):
            rdma = pltpu.make_async_remote_copy(
                src_ref=kv_buf.at[h],
                dst_ref=kv_buf.at[h + 1],
                send_sem=send_sems.at[h],
                recv_sem=recv_sems.at[h],
                device_id=(right,),
                device_id_type=pl.DeviceIdType.MESH,
            )
            rdma.start()
            rdma.wait()

        xb = x_ref[...].reshape(B * SQ, D_MODEL).astype(jnp.bfloat16)
        wq = wq_ref[...].astype(jnp.bfloat16)
        q = jnp.dot(xb, wq, preferred_element_type=jnp.float32)
        q = q.astype(jnp.bfloat16)

        for r in range(NR):
            kr = jnp.concatenate(
                [kv_buf[j, :, r * BLK:(r + 1) * BLK, 0:HD] for j in range(N_DEV)],
                axis=1)
            vr = jnp.concatenate(
                [kv_buf[j, :, r * BLK:(r + 1) * BLK, HD:2 * HD] for j in range(N_DEV)],
                axis=1)
            for b in range(B):
                for hh in range(HQ):
                    qs = q[b * SQ + r * BLK: b * SQ + (r + 1) * BLK,
                           hh * DH:(hh + 1) * DH]
                    ks = kr[b, :, hh * DH:(hh + 1) * DH]
                    vs = vr[b, :, hh * DH:(hh + 1) * DH]
                    s = lax.dot_general(
                        qs, ks, (((1,), (1,)), ((), ())),
                        preferred_element_type=jnp.float32) * 0.125
                    m = jnp.max(s, axis=1, keepdims=True)
                    w = jnp.exp(s - m)
                    w = w / jnp.sum(w, axis=1, keepdims=True)
                    ctx = jnp.dot(w.astype(jnp.bfloat16), vs,
                                  preferred_element_type=jnp.float32)
                    ctx_buf[b * SQ + r * BLK: b * SQ + (r + 1) * BLK,
                            hh * DH:(hh + 1) * DH] = ctx.astype(jnp.bfloat16)

        wo = wo_ref[...].astype(jnp.bfloat16)
        out = jnp.dot(ctx_buf[...], wo, preferred_element_type=jnp.float32)
        out_ref[...] = out.reshape(B, SQ, D_MODEL)

    return pl.pallas_call(
        body,
        out_shape=jax.ShapeDtypeStruct((B, SQ, D_MODEL), jnp.float32),
        in_specs=[pl.BlockSpec(memory_space=pltpu.VMEM)] * 5,
        out_specs=pl.BlockSpec(memory_space=pltpu.VMEM),
        scratch_shapes=[
            pltpu.VMEM((N_DEV, B, SQ, 2 * HD), jnp.bfloat16),
            pltpu.VMEM((B * SQ, HD), jnp.bfloat16),
            pltpu.SemaphoreType.DMA((N_DEV - 1,)),
            pltpu.SemaphoreType.DMA((N_DEV - 1,)),
        ],
        compiler_params=pltpu.CompilerParams(collective_id=0),
    )(x, Wq, K_ext, V_ext, Wo)


# baseline (device time: 92485 ns/iter reference)
import jax
import jax.numpy as jnp
from jax import lax
from jax.experimental import pallas as pl
from jax.experimental.pallas import tpu as pltpu

N_DEV = 16
B = 2
SQ = 256
D_MODEL = 512
HQ = 4
DH = 64
HD = HQ * DH
BLK = 64
NR = SQ // BLK
NFWD = 7
NBWD = 8


def kernel(x, Wq, K_ext, V_ext, Wo):
    def body(x_ref, wq_ref, k_ref, v_ref, wo_ref, out_ref,
             kvf, kvb, ctx_buf,
             fsend_sems, frecv_sems, bsend_sems, brecv_sems):
        my = lax.axis_index("i")
        left = lax.rem(my + N_DEV - 1, N_DEV)
        right = lax.rem(my + 1, N_DEV)

        barrier_sem = pltpu.get_barrier_semaphore()
        for nbr in (left, right):
            pl.semaphore_signal(
                barrier_sem, inc=1,
                device_id=(nbr,), device_id_type=pl.DeviceIdType.MESH,
            )
        pl.semaphore_wait(barrier_sem, 2)

        kvf[0, :, :, 0:HD] = k_ref[...].reshape(B, SQ, HD).astype(jnp.bfloat16)
        kvf[0, :, :, HD:2 * HD] = v_ref[...].reshape(B, SQ, HD).astype(jnp.bfloat16)

        q = None
        for h in range(NBWD):
            started = []
            if h < NFWD:
                fwd = pltpu.make_async_remote_copy(
                    src_ref=kvf.at[h],
                    dst_ref=kvf.at[h + 1],
                    send_sem=fsend_sems.at[h],
                    recv_sem=frecv_sems.at[h],
                    device_id=(right,),
                    device_id_type=pl.DeviceIdType.MESH,
                )
                fwd.start()
                started.append(fwd)
            bwd = pltpu.make_async_remote_copy(
                src_ref=kvf.at[0] if h == 0 else kvb.at[h - 1],
                dst_ref=kvb.at[h],
                send_sem=bsend_sems.at[h],
                recv_sem=brecv_sems.at[h],
                device_id=(left,),
                device_id_type=pl.DeviceIdType.MESH,
            )
            bwd.start()
            started.append(bwd)

            if h == 0:
                xb = x_ref[...].reshape(B * SQ, D_MODEL).astype(jnp.bfloat16)
                wq = wq_ref[...].astype(jnp.bfloat16)
                q = jnp.dot(xb, wq, preferred_element_type=jnp.float32)
                q = q.astype(jnp.bfloat16)

            for r in started:
                r.wait()

        for r in range(NR):
            rows = slice(r * BLK, (r + 1) * BLK)
            kr = jnp.concatenate(
                [kvf[j, :, rows, 0:HD] for j in range(NFWD + 1)]
                + [kvb[j, :, rows, 0:HD] for j in range(NBWD)],
                axis=1)
            vr = jnp.concatenate(
                [kvf[j, :, rows, HD:2 * HD] for j in range(NFWD + 1)]
                + [kvb[j, :, rows, HD:2 * HD] for j in range(NBWD)],
                axis=1)
            for b in range(B):
                for hh in range(HQ):
                    qs = q[b * SQ + r * BLK: b * SQ + (r + 1) * BLK,
                           hh * DH:(hh + 1) * DH]
                    ks = kr[b, :, hh * DH:(hh + 1) * DH]
                    vs = vr[b, :, hh * DH:(hh + 1) * DH]
                    s = lax.dot_general(
                        qs, ks, (((1,), (1,)), ((), ())),
                        preferred_element_type=jnp.float32) * 0.125
                    m = jnp.max(s, axis=1, keepdims=True)
                    w = jnp.exp(s - m)
                    w = w / jnp.sum(w, axis=1, keepdims=True)
                    ctx = jnp.dot(w.astype(jnp.bfloat16), vs,
                                  preferred_element_type=jnp.float32)
                    ctx_buf[b * SQ + r * BLK: b * SQ + (r + 1) * BLK,
                            hh * DH:(hh + 1) * DH] = ctx.astype(jnp.bfloat16)

        wo = wo_ref[...].astype(jnp.bfloat16)
        out = jnp.dot(ctx_buf[...], wo, preferred_element_type=jnp.float32)
        out_ref[...] = out.reshape(B, SQ, D_MODEL)

    return pl.pallas_call(
        body,
        out_shape=jax.ShapeDtypeStruct((B, SQ, D_MODEL), jnp.float32),
        in_specs=[pl.BlockSpec(memory_space=pltpu.VMEM)] * 5,
        out_specs=pl.BlockSpec(memory_space=pltpu.VMEM),
        scratch_shapes=[
            pltpu.VMEM((NFWD + 1, B, SQ, 2 * HD), jnp.bfloat16),
            pltpu.VMEM((NBWD, B, SQ, 2 * HD), jnp.bfloat16),
            pltpu.VMEM((B * SQ, HD), jnp.bfloat16),
            pltpu.SemaphoreType.DMA((NFWD,)),
            pltpu.SemaphoreType.DMA((NFWD,)),
            pltpu.SemaphoreType.DMA((NBWD,)),
            pltpu.SemaphoreType.DMA((NBWD,)),
        ],
        compiler_params=pltpu.CompilerParams(collective_id=0),
    )(x, Wq, K_ext, V_ext, Wo)


# device time: 65313 ns/iter; 1.4160x vs baseline; 1.4160x over previous
import jax
import jax.numpy as jnp
from jax import lax
from jax.experimental import pallas as pl
from jax.experimental.pallas import tpu as pltpu

N_DEV = 16
B = 2
SQ = 256
D_MODEL = 512
HQ = 4
DH = 64
HD = HQ * DH
BLK = 64
NR = SQ // BLK
NZ = 4
MESH = pl.DeviceIdType.MESH


def kernel(x, Wq, K_ext, V_ext, Wo):
    def body(x_ref, wq_ref, k_ref, v_ref, wo_ref, out_ref,
             zbuf, pbuf, ctx_buf,
             dsend, drecv, usend, urecv,
             r1a_s, r1a_r, r1b_s, r1b_r,
             r2a_s, r2a_r, r2b_s, r2b_r):
        my = lax.axis_index("i")
        my_z = my // NZ
        my_i = lax.rem(my, 4)
        right = my_z * 4 + lax.rem(my_i + 1, 4)
        left = my_z * 4 + lax.rem(my_i + 3, 4)
        up_dev = jnp.maximum(my - 4, 0)
        down_dev = jnp.minimum(my + 4, 15)

        def rdma(src, dst, ssem, rsem, dev):
            return pltpu.make_async_remote_copy(
                src_ref=src, dst_ref=dst, send_sem=ssem, recv_sem=rsem,
                device_id=(dev,), device_id_type=MESH)

        barrier_sem = pltpu.get_barrier_semaphore()
        for nbr in (left, right):
            pl.semaphore_signal(barrier_sem, inc=1, device_id=(nbr,),
                                device_id_type=MESH)

        @pl.when(my_z >= 1)
        def _():
            pl.semaphore_signal(barrier_sem, inc=1, device_id=(up_dev,),
                                device_id_type=MESH)

        @pl.when(my_z <= 2)
        def _():
            pl.semaphore_signal(barrier_sem, inc=1, device_id=(down_dev,),
                                device_id_type=MESH)

        is_mid = jnp.logical_and(my_z >= 1, my_z <= 2)

        @pl.when(is_mid)
        def _():
            pl.semaphore_wait(barrier_sem, 4)

        @pl.when(jnp.logical_not(is_mid))
        def _():
            pl.semaphore_wait(barrier_sem, 3)

        def start_r1(zz):
            rdma(zbuf.at[zz], pbuf.at[0 * 4 + zz],
                 r1a_s.at[zz], r1a_r.at[zz], right).start()
            rdma(zbuf.at[zz], pbuf.at[1 * 4 + zz],
                 r1b_s.at[zz], r1b_r.at[zz], left).start()

        for zz in range(NZ):
            @pl.when(my_z == zz)
            def _(zz=zz):
                zbuf[zz, :, :, 0:HD] = (
                    k_ref[...].reshape(B, SQ, HD).astype(jnp.bfloat16))
                zbuf[zz, :, :, HD:2 * HD] = (
                    v_ref[...].reshape(B, SQ, HD).astype(jnp.bfloat16))
                start_r1(zz)
                if zz <= 2:
                    rdma(zbuf.at[zz], zbuf.at[zz],
                         dsend.at[0], drecv.at[0], down_dev).start()
                if zz >= 1:
                    rdma(zbuf.at[zz], zbuf.at[zz],
                         usend.at[0], urecv.at[0], up_dev).start()

        xb = x_ref[...].reshape(B * SQ, D_MODEL).astype(jnp.bfloat16)
        wq = wq_ref[...].astype(jnp.bfloat16)
        q = jnp.dot(xb, wq, preferred_element_type=jnp.float32)
        q = q.astype(jnp.bfloat16)

        for zz in range(NZ):
            @pl.when(my_z == zz)
            def _(zz=zz):
                for h in range(NZ - 1):
                    if h >= 1:
                        if zz <= 2 and h <= zz:
                            rdma(zbuf.at[zz - h], zbuf.at[zz - h],
                                 dsend.at[h], drecv.at[h], down_dev).start()
                        if zz >= 1 and zz + h <= 3:
                            rdma(zbuf.at[zz + h], zbuf.at[zz + h],
                                 usend.at[h], urecv.at[h], up_dev).start()
                    if zz >= h + 1:
                        slot = zz - h - 1
                        rdma(zbuf.at[slot], zbuf.at[slot],
                             dsend.at[h], drecv.at[h], down_dev).wait_recv()
                        start_r1(slot)
                    if zz <= 2 - h:
                        slot = zz + h + 1
                        rdma(zbuf.at[slot], zbuf.at[slot],
                             usend.at[h], urecv.at[h], up_dev).wait_recv()
                        start_r1(slot)

        for zz in range(NZ):
            rdma(zbuf.at[zz], pbuf.at[0 * 4 + zz],
                 r1a_s.at[zz], r1a_r.at[zz], right).wait_recv()
            if zz < 2:
                rdma(pbuf.at[0 * 4 + zz], pbuf.at[2 * 4 + zz],
                     r2a_s.at[zz], r2a_r.at[zz], right).start()
            rdma(zbuf.at[zz], pbuf.at[1 * 4 + zz],
                 r1b_s.at[zz], r1b_r.at[zz], left).wait_recv()
            if zz >= 2:
                rdma(pbuf.at[1 * 4 + zz], pbuf.at[2 * 4 + zz],
                     r2b_s.at[zz - 2], r2b_r.at[zz - 2], left).start()

        for j in range(2):
            rdma(pbuf.at[0 * 4 + j], pbuf.at[2 * 4 + j],
                 r2a_s.at[j], r2a_r.at[j], right).wait_recv()
            rdma(pbuf.at[1 * 4 + j + 2], pbuf.at[2 * 4 + j + 2],
                 r2b_s.at[j], r2b_r.at[j], left).wait_recv()

        for zz in range(NZ):
            rdma(zbuf.at[zz], pbuf.at[0 * 4 + zz],
                 r1a_s.at[zz], r1a_r.at[zz], right).wait_send()
            rdma(zbuf.at[zz], pbuf.at[1 * 4 + zz],
                 r1b_s.at[zz], r1b_r.at[zz], left).wait_send()
        for j in range(2):
            rdma(pbuf.at[0 * 4 + j], pbuf.at[2 * 4 + j],
                 r2a_s.at[j], r2a_r.at[j], right).wait_send()
            rdma(pbuf.at[1 * 4 + j + 2], pbuf.at[2 * 4 + j + 2],
                 r2b_s.at[j], r2b_r.at[j], left).wait_send()
        for zz in range(NZ):
            @pl.when(my_z == zz)
            def _(zz=zz):
                for h in range(NZ - 1):
                    if zz <= 2 and h <= zz:
                        rdma(zbuf.at[zz - h], zbuf.at[zz - h],
                             dsend.at[h], drecv.at[h], down_dev).wait_send()
                    if zz >= 1 and zz + h <= 3:
                        rdma(zbuf.at[zz + h], zbuf.at[zz + h],
                             usend.at[h], urecv.at[h], up_dev).wait_send()

        for r in range(NR):
            rows = slice(r * BLK, (r + 1) * BLK)
            kr = jnp.concatenate(
                [zbuf[zz, :, rows, 0:HD] for zz in range(NZ)]
                + [pbuf[s, :, rows, 0:HD] for s in range(12)],
                axis=1)
            vr = jnp.concatenate(
                [zbuf[zz, :, rows, HD:2 * HD] for zz in range(NZ)]
                + [pbuf[s, :, rows, HD:2 * HD] for s in range(12)],
                axis=1)
            for b in range(B):
                for hh in range(HQ):
                    qs = q[b * SQ + r * BLK: b * SQ + (r + 1) * BLK,
                           hh * DH:(hh + 1) * DH]
                    ks = kr[b, :, hh * DH:(hh + 1) * DH]
                    vs = vr[b, :, hh * DH:(hh + 1) * DH]
                    s = lax.dot_general(
                        qs, ks, (((1,), (1,)), ((), ())),
                        preferred_element_type=jnp.float32) * 0.125
                    m = jnp.max(s, axis=1, keepdims=True)
                    w = jnp.exp(s - m)
                    w = w / jnp.sum(w, axis=1, keepdims=True)
                    ctx = jnp.dot(w.astype(jnp.bfloat16), vs,
                                  preferred_element_type=jnp.float32)
                    ctx_buf[b * SQ + r * BLK: b * SQ + (r + 1) * BLK,
                            hh * DH:(hh + 1) * DH] = ctx.astype(jnp.bfloat16)

        wo = wo_ref[...].astype(jnp.bfloat16)
        out = jnp.dot(ctx_buf[...], wo, preferred_element_type=jnp.float32)
        out_ref[...] = out.reshape(B, SQ, D_MODEL)

    return pl.pallas_call(
        body,
        out_shape=jax.ShapeDtypeStruct((B, SQ, D_MODEL), jnp.float32),
        in_specs=[pl.BlockSpec(memory_space=pltpu.VMEM)] * 5,
        out_specs=pl.BlockSpec(memory_space=pltpu.VMEM),
        scratch_shapes=[
            pltpu.VMEM((NZ, B, SQ, 2 * HD), jnp.bfloat16),
            pltpu.VMEM((12, B, SQ, 2 * HD), jnp.bfloat16),
            pltpu.VMEM((B * SQ, HD), jnp.bfloat16),
            pltpu.SemaphoreType.DMA((3,)),
            pltpu.SemaphoreType.DMA((3,)),
            pltpu.SemaphoreType.DMA((3,)),
            pltpu.SemaphoreType.DMA((3,)),
            pltpu.SemaphoreType.DMA((4,)),
            pltpu.SemaphoreType.DMA((4,)),
            pltpu.SemaphoreType.DMA((4,)),
            pltpu.SemaphoreType.DMA((4,)),
            pltpu.SemaphoreType.DMA((2,)),
            pltpu.SemaphoreType.DMA((2,)),
            pltpu.SemaphoreType.DMA((2,)),
            pltpu.SemaphoreType.DMA((2,)),
        ],
        compiler_params=pltpu.CompilerParams(collective_id=0),
    )(x, Wq, K_ext, V_ext, Wo)


# device time: 58459 ns/iter; 1.5820x vs baseline; 1.1172x over previous
import jax
import jax.numpy as jnp
from jax import lax
from jax.experimental import pallas as pl
from jax.experimental.pallas import tpu as pltpu

N_DEV = 16
B = 2
SQ = 256
D_MODEL = 512
HQ = 4
DH = 64
HD = HQ * DH
BLK = 64
NR = SQ // BLK
NZ = 4
CK = NZ * BLK
NCOMBO = NR * B * HQ
MESH = pl.DeviceIdType.MESH


def kernel(x, Wq, K_ext, V_ext, Wo):
    def body(x_ref, wq_ref, k_ref, v_ref, wo_ref, out_ref,
             zbuf, pbuf, sbuf, ctx_buf,
             dsend, drecv, usend, urecv,
             r1a_s, r1a_r, r1b_s, r1b_r,
             r2a_s, r2a_r, r2b_s, r2b_r):
        my = lax.axis_index("i")
        my_z = my // NZ
        my_i = lax.rem(my, 4)
        right = my_z * 4 + lax.rem(my_i + 1, 4)
        left = my_z * 4 + lax.rem(my_i + 3, 4)
        up_dev = jnp.maximum(my - 4, 0)
        down_dev = jnp.minimum(my + 4, 15)

        def rdma(src, dst, ssem, rsem, dev):
            return pltpu.make_async_remote_copy(
                src_ref=src, dst_ref=dst, send_sem=ssem, recv_sem=rsem,
                device_id=(dev,), device_id_type=MESH)

        barrier_sem = pltpu.get_barrier_semaphore()
        for nbr in (left, right):
            pl.semaphore_signal(barrier_sem, inc=1, device_id=(nbr,),
                                device_id_type=MESH)

        @pl.when(my_z >= 1)
        def _():
            pl.semaphore_signal(barrier_sem, inc=1, device_id=(up_dev,),
                                device_id_type=MESH)

        @pl.when(my_z <= 2)
        def _():
            pl.semaphore_signal(barrier_sem, inc=1, device_id=(down_dev,),
                                device_id_type=MESH)

        is_mid = jnp.logical_and(my_z >= 1, my_z <= 2)

        @pl.when(is_mid)
        def _():
            pl.semaphore_wait(barrier_sem, 4)

        @pl.when(jnp.logical_not(is_mid))
        def _():
            pl.semaphore_wait(barrier_sem, 3)

        def start_r1(zz):
            rdma(zbuf.at[zz], pbuf.at[0 * 4 + zz],
                 r1a_s.at[zz], r1a_r.at[zz], right).start()
            rdma(zbuf.at[zz], pbuf.at[1 * 4 + zz],
                 r1b_s.at[zz], r1b_r.at[zz], left).start()

        for zz in range(NZ):
            @pl.when(my_z == zz)
            def _(zz=zz):
                zbuf[zz, :, :, 0:HD] = (
                    k_ref[...].reshape(B, SQ, HD).astype(jnp.bfloat16))
                zbuf[zz, :, :, HD:2 * HD] = (
                    v_ref[...].reshape(B, SQ, HD).astype(jnp.bfloat16))
                start_r1(zz)
                if zz <= 2:
                    rdma(zbuf.at[zz], zbuf.at[zz],
                         dsend.at[0], drecv.at[0], down_dev).start()
                if zz >= 1:
                    rdma(zbuf.at[zz], zbuf.at[zz],
                         usend.at[0], urecv.at[0], up_dev).start()

        xb = x_ref[...].reshape(B * SQ, D_MODEL).astype(jnp.bfloat16)
        wq = wq_ref[...].astype(jnp.bfloat16)
        q = jnp.dot(xb, wq, preferred_element_type=jnp.float32)
        q = q.astype(jnp.bfloat16)

        def q_slice(r, b, hh):
            return q[b * SQ + r * BLK: b * SQ + (r + 1) * BLK,
                     hh * DH:(hh + 1) * DH]

        def score_col(c, pieces):
            for r in range(NR):
                rows = slice(r * BLK, (r + 1) * BLK)
                kcol = jnp.concatenate(
                    [buf[s, :, rows, 0:HD] for (buf, s) in pieces],
                    axis=1)
                for b in range(B):
                    for hh in range(HQ):
                        ks = kcol[b, :, hh * DH:(hh + 1) * DH]
                        sc = lax.dot_general(
                            q_slice(r, b, hh), ks, (((1,), (1,)), ((), ())),
                            preferred_element_type=jnp.float32) * 0.125
                        idx = (r * B + b) * HQ + hh
                        sbuf[idx * BLK:(idx + 1) * BLK,
                             c * CK:(c + 1) * CK] = sc

        for zz in range(NZ):
            @pl.when(my_z == zz)
            def _(zz=zz):
                for h in range(NZ - 1):
                    if h >= 1:
                        if zz <= 2 and h <= zz:
                            rdma(zbuf.at[zz - h], zbuf.at[zz - h],
                                 dsend.at[h], drecv.at[h], down_dev).start()
                        if zz >= 1 and zz + h <= 3:
                            rdma(zbuf.at[zz + h], zbuf.at[zz + h],
                                 usend.at[h], urecv.at[h], up_dev).start()
                    if zz >= h + 1:
                        slot = zz - h - 1
                        rdma(zbuf.at[slot], zbuf.at[slot],
                             dsend.at[h], drecv.at[h], down_dev).wait_recv()
                        start_r1(slot)
                    if zz <= 2 - h:
                        slot = zz + h + 1
                        rdma(zbuf.at[slot], zbuf.at[slot],
                             usend.at[h], urecv.at[h], up_dev).wait_recv()
                        start_r1(slot)

        score_col(0, [(zbuf, zz) for zz in range(NZ)])

        for zz in range(NZ):
            rdma(zbuf.at[zz], pbuf.at[0 * 4 + zz],
                 r1a_s.at[zz], r1a_r.at[zz], right).wait_recv()
            if zz < 2:
                rdma(pbuf.at[0 * 4 + zz], pbuf.at[2 * 4 + zz],
                     r2a_s.at[zz], r2a_r.at[zz], right).start()
            rdma(zbuf.at[zz], pbuf.at[1 * 4 + zz],
                 r1b_s.at[zz], r1b_r.at[zz], left).wait_recv()
            if zz >= 2:
                rdma(pbuf.at[1 * 4 + zz], pbuf.at[2 * 4 + zz],
                     r2b_s.at[zz - 2], r2b_r.at[zz - 2], left).start()

        score_col(1, [(pbuf, 0 * 4 + zz) for zz in range(NZ)])
        score_col(2, [(pbuf, 1 * 4 + zz) for zz in range(NZ)])

        for j in range(2):
            rdma(pbuf.at[0 * 4 + j], pbuf.at[2 * 4 + j],
                 r2a_s.at[j], r2a_r.at[j], right).wait_recv()
            rdma(pbuf.at[1 * 4 + j + 2], pbuf.at[2 * 4 + j + 2],
                 r2b_s.at[j], r2b_r.at[j], left).wait_recv()

        score_col(3, [(pbuf, 2 * 4 + zz) for zz in range(NZ)])

        for zz in range(NZ):
            rdma(zbuf.at[zz], pbuf.at[0 * 4 + zz],
                 r1a_s.at[zz], r1a_r.at[zz], right).wait_send()
            rdma(zbuf.at[zz], pbuf.at[1 * 4 + zz],
                 r1b_s.at[zz], r1b_r.at[zz], left).wait_send()
        for j in range(2):
            rdma(pbuf.at[0 * 4 + j], pbuf.at[2 * 4 + j],
                 r2a_s.at[j], r2a_r.at[j], right).wait_send()
            rdma(pbuf.at[1 * 4 + j + 2], pbuf.at[2 * 4 + j + 2],
                 r2b_s.at[j], r2b_r.at[j], left).wait_send()
        for zz in range(NZ):
            @pl.when(my_z == zz)
            def _(zz=zz):
                for h in range(NZ - 1):
                    if zz <= 2 and h <= zz:
                        rdma(zbuf.at[zz - h], zbuf.at[zz - h],
                             dsend.at[h], drecv.at[h], down_dev).wait_send()
                    if zz >= 1 and zz + h <= 3:
                        rdma(zbuf.at[zz + h], zbuf.at[zz + h],
                             usend.at[h], urecv.at[h], up_dev).wait_send()

        vbufs = [zbuf, pbuf, pbuf, pbuf]
        voffs = [0, 0, 4, 8]
        for r in range(NR):
            rows = slice(r * BLK, (r + 1) * BLK)
            vcols = [
                jnp.concatenate(
                    [vbufs[c][voffs[c] + zz, :, rows, HD:2 * HD]
                     for zz in range(NZ)], axis=1)
                for c in range(4)
            ]
            for b in range(B):
                for hh in range(HQ):
                    idx = (r * B + b) * HQ + hh
                    s = sbuf[idx * BLK:(idx + 1) * BLK, :]
                    m = jnp.max(s, axis=1, keepdims=True)
                    w = jnp.exp(s - m)
                    w = (w / jnp.sum(w, axis=1, keepdims=True)
                         ).astype(jnp.bfloat16)
                    ctx = sum(
                        jnp.dot(w[:, c * CK:(c + 1) * CK],
                                vcols[c][b, :, hh * DH:(hh + 1) * DH],
                                preferred_element_type=jnp.float32)
                        for c in range(4))
                    ctx_buf[b * SQ + r * BLK: b * SQ + (r + 1) * BLK,
                            hh * DH:(hh + 1) * DH] = ctx.astype(jnp.bfloat16)

        wo = wo_ref[...].astype(jnp.bfloat16)
        out = jnp.dot(ctx_buf[...], wo, preferred_element_type=jnp.float32)
        out_ref[...] = out.reshape(B, SQ, D_MODEL)

    return pl.pallas_call(
        body,
        out_shape=jax.ShapeDtypeStruct((B, SQ, D_MODEL), jnp.float32),
        in_specs=[pl.BlockSpec(memory_space=pltpu.VMEM)] * 5,
        out_specs=pl.BlockSpec(memory_space=pltpu.VMEM),
        scratch_shapes=[
            pltpu.VMEM((NZ, B, SQ, 2 * HD), jnp.bfloat16),
            pltpu.VMEM((12, B, SQ, 2 * HD), jnp.bfloat16),
            pltpu.VMEM((NCOMBO * BLK, 4 * CK), jnp.float32),
            pltpu.VMEM((B * SQ, HD), jnp.bfloat16),
            pltpu.SemaphoreType.DMA((3,)),
            pltpu.SemaphoreType.DMA((3,)),
            pltpu.SemaphoreType.DMA((3,)),
            pltpu.SemaphoreType.DMA((3,)),
            pltpu.SemaphoreType.DMA((4,)),
            pltpu.SemaphoreType.DMA((4,)),
            pltpu.SemaphoreType.DMA((4,)),
            pltpu.SemaphoreType.DMA((4,)),
            pltpu.SemaphoreType.DMA((2,)),
            pltpu.SemaphoreType.DMA((2,)),
            pltpu.SemaphoreType.DMA((2,)),
            pltpu.SemaphoreType.DMA((2,)),
        ],
        compiler_params=pltpu.CompilerParams(collective_id=0),
    )(x, Wq, K_ext, V_ext, Wo)
